# Initial kernel scaffold; baseline (speedup 1.0000x reference)
import jax
import jax.numpy as jnp
from jax.experimental import pallas as pl


def kernel(token_ids, table):
    raise NotImplementedError("write your pallas kernel here")


# speedup vs baseline: 1.0508x; 1.0508x over previous
import functools

import jax
import jax.numpy as jnp
from jax import lax
from jax.experimental import pallas as pl
from jax.experimental.pallas import tpu as pltpu
from jax.experimental.pallas import tpu_sc as plsc

B, L, N, D = 4096, 50, 4, 64
P = B * L
NC, NS = 2, 16
NW = NC * NS
P_PER_W = P // NW
CHUNK = 128
NCHUNK = P_PER_W // CHUNK
LANES = 16


def _body(table_hbm, idx_hbm, out_hbm, idx_v, r0, r1, r2, r3, inv_v,
          out_v, sem):
    wid = lax.axis_index("s") * NC + lax.axis_index("c")
    base = wid * P_PER_W
    rows = (r0, r1, r2, r3)

    def chunk_body(g, carry):
        pltpu.sync_copy(idx_hbm.at[wid, g], idx_v)
        copies = [
            pltpu.async_copy(table_hbm.at[idx_v.at[j]], rows[j], sem)
            for j in range(N)
        ]
        for s in range(CHUNK // LANES):
            sl = pl.ds(s * LANES, LANES)
            cnt = jnp.zeros((LANES,), jnp.int32)
            for j in range(N):
                cnt = cnt + jnp.where(idx_v[j, sl] != 0, 1, 0)
            inv = jnp.where(
                cnt == 0, 0.0,
                jnp.where(cnt == 1, 1.0,
                          jnp.where(cnt == 2, 0.5,
                                    jnp.where(cnt == 3, 1.0 / 3.0, 0.25))))
            inv_v[sl] = inv.astype(jnp.float32)
        for c in copies:
            c.wait()

        def grp_body(s, carry2):
            inv16 = inv_v[pl.ds(s * LANES, LANES)]
            for i in range(LANES):
                p = s * LANES + i
                invp = jnp.broadcast_to(inv16[i], (LANES,))
                for d in range(D // LANES):
                    dsl = pl.ds(d * LANES, LANES)
                    acc = r0[p, dsl] + r1[p, dsl] + r2[p, dsl] + r3[p, dsl]
                    out_v[p, dsl] = acc * invp
            return carry2

        lax.fori_loop(0, CHUNK // LANES, grp_body, 0)
        pltpu.sync_copy(out_v, out_hbm.at[pl.ds(base + g * CHUNK, CHUNK)])
        return carry

    lax.fori_loop(0, NCHUNK, chunk_body, 0)


@jax.jit
def kernel(token_ids, table):
    ids = token_ids.reshape(P, N).T.reshape(N, NW, NCHUNK, CHUNK)
    ids = ids.transpose(1, 2, 0, 3)

    mesh = plsc.VectorSubcoreMesh(core_axis_name="c", subcore_axis_name="s")
    out = pl.kernel(
        _body,
        out_type=jax.ShapeDtypeStruct((P, D), jnp.float32),
        mesh=mesh,
        compiler_params=pltpu.CompilerParams(use_tc_tiling_on_sc=False),
        scratch_types=[
            pltpu.VMEM((N, CHUNK), jnp.int32),
            pltpu.VMEM((CHUNK, D), jnp.float32),
            pltpu.VMEM((CHUNK, D), jnp.float32),
            pltpu.VMEM((CHUNK, D), jnp.float32),
            pltpu.VMEM((CHUNK, D), jnp.float32),
            pltpu.VMEM((CHUNK,), jnp.float32),
            pltpu.VMEM((CHUNK, D), jnp.float32),
            pltpu.SemaphoreType.DMA,
        ],
    )(table, ids)
    return out.reshape(B, L, D)


# speedup vs baseline: 1.1336x; 1.0788x over previous
import functools

import jax
import jax.numpy as jnp
from jax import lax
from jax.experimental import pallas as pl
from jax.experimental.pallas import tpu as pltpu
from jax.experimental.pallas import tpu_sc as plsc

B, L, N, D = 4096, 50, 4, 64
P = B * L
NC, NS = 2, 16
NW = NC * NS
P_PER_W = P // NW
CHUNK = 128
NCHUNK = P_PER_W // CHUNK
LANES = 16


def _body(table_hbm, idx_hbm, out_hbm, idx_v, rows_v, inv_v, out_v,
          sem0, sem1):
    wid = lax.axis_index("s") * NC + lax.axis_index("c")
    base = wid * P_PER_W
    sems = (sem0, sem1)

    def fire(g, slot):
        pltpu.sync_copy(idx_hbm.at[wid, g], idx_v.at[slot])
        for j in range(N):
            pltpu.async_copy(table_hbm.at[idx_v.at[slot, j]],
                             rows_v.at[slot, j], sems[slot])

    def drain(slot):
        for j in range(N):
            pltpu.make_async_copy(table_hbm.at[idx_v.at[slot, j]],
                                  rows_v.at[slot, j], sems[slot]).wait()

    def process(g, slot):
        for s in range(CHUNK // LANES):
            sl = pl.ds(s * LANES, LANES)
            cnt = jnp.zeros((LANES,), jnp.int32)
            for j in range(N):
                cnt = cnt + jnp.where(idx_v[slot, j, sl] != 0, 1, 0)
            inv = jnp.where(
                cnt == 0, 0.0,
                jnp.where(cnt == 1, 1.0,
                          jnp.where(cnt == 2, 0.5,
                                    jnp.where(cnt == 3, 1.0 / 3.0, 0.25))))
            inv_v[sl] = inv.astype(jnp.float32)
        drain(slot)

        def grp_body(s, carry2):
            inv16 = inv_v[pl.ds(s * LANES, LANES)]
            for i in range(LANES):
                p = s * LANES + i
                invp = jnp.broadcast_to(inv16[i], (LANES,))
                for d in range(D // LANES):
                    dsl = pl.ds(d * LANES, LANES)
                    acc = (rows_v[slot, 0, p, dsl] + rows_v[slot, 1, p, dsl]
                           + rows_v[slot, 2, p, dsl]
                           + rows_v[slot, 3, p, dsl])
                    out_v[p, dsl] = acc * invp
            return carry2

        lax.fori_loop(0, CHUNK // LANES, grp_body, 0)
        pltpu.sync_copy(out_v, out_hbm.at[pl.ds(base + g * CHUNK, CHUNK)])

    fire(0, 0)

    def chunk_pair(it, carry):
        for sub in range(2):
            g = 2 * it + sub

            @pl.when(g + 1 < NCHUNK)
            def _():
                fire(g + 1, 1 - sub)

            process(g, sub)
        return carry

    lax.fori_loop(0, NCHUNK // 2, chunk_pair, 0)


@jax.jit
def kernel(token_ids, table):
    ids = token_ids.reshape(P, N).T.reshape(N, NW, NCHUNK, CHUNK)
    ids = ids.transpose(1, 2, 0, 3)

    mesh = plsc.VectorSubcoreMesh(core_axis_name="c", subcore_axis_name="s")
    out = pl.kernel(
        _body,
        out_type=jax.ShapeDtypeStruct((P, D), jnp.float32),
        mesh=mesh,
        compiler_params=pltpu.CompilerParams(use_tc_tiling_on_sc=False),
        scratch_types=[
            pltpu.VMEM((2, N, CHUNK), jnp.int32),
            pltpu.VMEM((2, N, CHUNK, D), jnp.float32),
            pltpu.VMEM((CHUNK,), jnp.float32),
            pltpu.VMEM((CHUNK, D), jnp.float32),
            pltpu.SemaphoreType.DMA,
            pltpu.SemaphoreType.DMA,
        ],
    )(table, ids)
    return out.reshape(B, L, D)
